# baseline (speedup 1.0000x reference)
import jax
import jax.numpy as jnp
from jax import lax
from jax.experimental import pallas as pl
from jax.experimental.pallas import tpu as pltpu
from jax.experimental.pallas import tpu_sc as plsc

BATCH = 4
SEQ = 2048
D_MODEL = 1024

_info = plsc.get_sparse_core_info()
NC, NS = _info.num_cores, _info.num_subcores
NW = NC * NS
POS_PER_W = SEQ // NW
CHUNK = 16
NPC = POS_PER_W // CHUNK
NROUND = NPC * BATCH
NB = 3


def _emb_kernel(tok_hbm, wte_hbm, wpe_hbm, out_hbm,
                idx_v, wpe_v, acc0, acc1, acc2,
                gsem0, gsem1, gsem2, ssem0, ssem1, ssem2, isem, wsem):
    wid = lax.axis_index("s") * NC + lax.axis_index("c")
    pos0 = wid * POS_PER_W
    acc = (acc0, acc1, acc2)
    gsem = (gsem0, gsem1, gsem2)
    ssem = (ssem0, ssem1, ssem2)

    idx_descs = [
        pltpu.async_copy(tok_hbm.at[pl.ds(b * SEQ + pos0, POS_PER_W)],
                         idx_v.at[b], isem)
        for b in range(BATCH)
    ]
    wpe_descs = [
        pltpu.async_copy(wpe_hbm.at[pl.ds(pos0 + pc * CHUNK, CHUNK)],
                         wpe_v.at[pl.ds(pc * CHUNK, CHUNK)], wsem)
        for pc in range(NPC)
    ]
    for d in idx_descs:
        d.wait()

    def gather(r):
        pc, b = divmod(r, BATCH)
        return pltpu.async_copy(
            wte_hbm.at[idx_v.at[b, pl.ds(pc * CHUNK, CHUNK)]],
            acc[r % NB], gsem[r % NB])

    g_descs = {r: gather(r) for r in range(NB - 1)}
    s_descs = {}
    for r in range(NROUND):
        buf = r % NB
        pc, b = divmod(r, BATCH)
        if b == 0:
            wpe_descs[pc].wait()
        g_descs[r].wait()
        a = acc[buf]
        w0 = pc * CHUNK

        base = b * SEQ + pos0 + pc * CHUNK
        half = CHUNK // 2

        def make_col_body(h):
            def col_body(c):
                for row in range(h * half, (h + 1) * half):
                    x = wpe_v[w0 + row, pl.ds(c, 16)]
                    plsc.addupdate(a.at[row, pl.ds(c, 16)], x)
            return col_body

        plsc.parallel_loop(0, D_MODEL, step=16)(make_col_body(0))
        s0 = pltpu.async_copy(
            a.at[pl.ds(0, half)], out_hbm.at[pl.ds(base, half)], ssem[buf])
        plsc.parallel_loop(0, D_MODEL, step=16)(make_col_body(1))
        s1 = pltpu.async_copy(
            a.at[pl.ds(half, half)], out_hbm.at[pl.ds(base + half, half)],
            ssem[buf])
        s_descs[r] = (s0, s1)
        if r + NB - 1 < NROUND:
            if r - 1 in s_descs:
                for d in s_descs[r - 1]:
                    d.wait()
            g_descs[r + NB - 1] = gather(r + NB - 1)
    for r in range(NROUND - NB, NROUND):
        for d in s_descs[r]:
            d.wait()


@jax.jit
def _run(tok_flat, wte, wpe):
    mesh = plsc.VectorSubcoreMesh(core_axis_name="c", subcore_axis_name="s")
    f = pl.kernel(
        _emb_kernel,
        out_type=jax.ShapeDtypeStruct((BATCH * SEQ, D_MODEL), jnp.float32),
        mesh=mesh,
        scratch_types=[
            pltpu.VMEM((BATCH, POS_PER_W), jnp.int32),
            pltpu.VMEM((POS_PER_W, D_MODEL), jnp.float32),
            pltpu.VMEM((CHUNK, D_MODEL), jnp.float32),
            pltpu.VMEM((CHUNK, D_MODEL), jnp.float32),
            pltpu.VMEM((CHUNK, D_MODEL), jnp.float32),
            pltpu.SemaphoreType.DMA,
            pltpu.SemaphoreType.DMA,
            pltpu.SemaphoreType.DMA,
            pltpu.SemaphoreType.DMA,
            pltpu.SemaphoreType.DMA,
            pltpu.SemaphoreType.DMA,
            pltpu.SemaphoreType.DMA,
            pltpu.SemaphoreType.DMA,
        ],
    )
    return f(tok_flat, wte, wpe)


def kernel(tokens, wte, wpe):
    tok_flat = tokens.reshape(-1).astype(jnp.int32)
    out = _run(tok_flat, wte, wpe)
    return out.reshape(BATCH, SEQ, D_MODEL)
